# baseline (speedup 1.0000x reference)
import jax
import jax.numpy as jnp
from jax import lax
from jax.experimental import pallas as pl
from jax.experimental.pallas import tpu as pltpu
from jax.experimental.pallas import tpu_sc as plsc

_B = 16384
_XW = 16
_DIMS = (3, 4, 6, 4, 50)
_OUT_W = _XW + sum(_DIMS)

_NC, _NS = 2, 16
_NW = _NC * _NS
_BPW = _B // _NW
_NCHK = _BPW // 16
_QCHK = _NCHK // 4
_QW = _BPW // 4

_TBASE = []
_a = 0
for _v, _d in zip((6, 7, 12, 7, 96), _DIMS):
    _TBASE.append(_a)
    _a += -(-(_v * _d) // 8) * 8
_TPACK = _a

_COL_OFF = []
_o = _XW
for _d in _DIMS:
    _COL_OFF.append(_o)
    _o += _d


def _body(xt_hbm, y_hbm, tbl_hbm, out_hbm, tbl_v, y_v, stage_v, in_sem,
          x_sem, st_sem):
    wid = lax.axis_index("s") * _NC + lax.axis_index("c")
    base = wid * _BPW

    c1 = pltpu.make_async_copy(tbl_hbm, tbl_v, in_sem)
    c2 = pltpu.make_async_copy(y_hbm.at[pl.ds(base, _BPW), :], y_v, in_sem)
    c3 = pltpu.make_async_copy(
        xt_hbm.at[:, pl.ds(base, _BPW)],
        stage_v.at[pl.ds(0, _XW), :], x_sem)
    c1.start()
    c2.start()
    c3.start()
    c1.wait()
    c2.wait()

    iota = jax.lax.iota(jnp.int32, 16)

    def chunk(c, carry):
        r0 = c * 16
        rows = r0 + iota
        for t in range(5):
            d = _DIMS[t]
            yt = plsc.load_gather(y_v, [rows, jnp.full((16,), t, jnp.int32)])
            srcb = yt * d + _TBASE[t]
            for j in range(d):
                v = plsc.load_gather(tbl_v, [srcb + j])
                stage_v[_COL_OFF[t] + j, pl.ds(r0, 16)] = v
        return carry

    lax.fori_loop(0, _QCHK, chunk, 0)
    c3.wait()
    stores = []
    for q in range(4):
        if q:
            lax.fori_loop(q * _QCHK, (q + 1) * _QCHK, chunk, 0)
        s = pltpu.make_async_copy(
            stage_v.at[:, pl.ds(q * _QW, _QW)],
            out_hbm.at[:, pl.ds(base + q * _QW, _QW)], st_sem)
        s.start()
        stores.append(s)
    for s in stores:
        s.wait()


def kernel(x, y, emb0, emb1, emb2, emb3, emb4):
    xt = x.T
    pieces = []
    for e, b, nb in zip((emb0, emb1, emb2, emb3, emb4),
                        _TBASE, _TBASE[1:] + [_TPACK]):
        r = e.reshape(-1)
        pieces.append(r)
        pad = nb - b - r.shape[0]
        if pad:
            pieces.append(jnp.zeros((pad,), jnp.float32))
    tbl = jnp.concatenate(pieces)

    mesh = plsc.VectorSubcoreMesh(core_axis_name="c", subcore_axis_name="s")
    kern = pl.kernel(
        _body,
        out_type=jax.ShapeDtypeStruct((_OUT_W, _B), jnp.float32),
        mesh=mesh,
        scratch_types=[
            pltpu.VMEM((_TPACK,), jnp.float32),
            pltpu.VMEM((_BPW, 5), jnp.int32),
            pltpu.VMEM((_OUT_W, _BPW), jnp.float32),
            pltpu.SemaphoreType.DMA,
            pltpu.SemaphoreType.DMA,
            pltpu.SemaphoreType.DMA,
        ],
        compiler_params=pltpu.CompilerParams(
            needs_layout_passes=False, use_tc_tiling_on_sc=False),
    )
    out_t = kern(xt, y, tbl)
    return out_t.T


# speedup vs baseline: 1.2005x; 1.2005x over previous
import jax
import jax.numpy as jnp
from jax import lax
from jax.experimental import pallas as pl
from jax.experimental.pallas import tpu as pltpu
from jax.experimental.pallas import tpu_sc as plsc

_B = 16384
_XW = 16
_DIMS = (3, 4, 6, 4, 50)
_OUT_W = _XW + sum(_DIMS)

_NC, _NS = 2, 16
_NW = _NC * _NS
_BPW = _B // _NW
_NCHK = _BPW // 16
_QCHK = _NCHK // 4
_QW = _BPW // 4

_TBASE = []
_a = 0
for _v, _d in zip((6, 7, 12, 7, 96), _DIMS):
    _TBASE.append(_a)
    _a += -(-(_v * _d) // 8) * 8
_TPACK = _a

_COL_OFF = []
_o = _XW
for _d in _DIMS:
    _COL_OFF.append(_o)
    _o += _d


def _body(xt_hbm, y_hbm, tbl_hbm, out_hbm, tbl_v, y_v, stage_v, in_sem,
          x_sem, st_sem):
    wid = lax.axis_index("s") * _NC + lax.axis_index("c")
    base = wid * _BPW

    c1 = pltpu.make_async_copy(tbl_hbm, tbl_v, in_sem)
    c2 = pltpu.make_async_copy(
        y_hbm.at[pl.ds(wid * 5 * _BPW, 5 * _BPW)], y_v, in_sem)
    c3 = pltpu.make_async_copy(
        xt_hbm.at[:, pl.ds(base, _BPW)],
        stage_v.at[pl.ds(0, _XW), :], x_sem)
    c1.start()
    c2.start()
    c3.start()
    c1.wait()
    c2.wait()

    iota = jax.lax.iota(jnp.int32, 16)

    def chunk(c, carry):
        r0 = c * 16
        for t in range(5):
            d = _DIMS[t]
            yt = y_v[pl.ds(t * _BPW + r0, 16)]
            srcb = yt * d + _TBASE[t]
            for j in range(d):
                v = plsc.load_gather(tbl_v, [srcb + j])
                stage_v[_COL_OFF[t] + j, pl.ds(r0, 16)] = v
        return carry

    lax.fori_loop(0, _QCHK, chunk, 0)
    c3.wait()
    stores = []
    for q in range(4):
        if q:
            lax.fori_loop(q * _QCHK, (q + 1) * _QCHK, chunk, 0)
        s = pltpu.make_async_copy(
            stage_v.at[:, pl.ds(q * _QW, _QW)],
            out_hbm.at[:, pl.ds(base + q * _QW, _QW)], st_sem)
        s.start()
        stores.append(s)
    for s in stores:
        s.wait()


def kernel(x, y, emb0, emb1, emb2, emb3, emb4):
    yw = y.T.reshape(5, _NW, _BPW).transpose(1, 0, 2).reshape(-1)
    xt = x.T
    pieces = []
    for e, b, nb in zip((emb0, emb1, emb2, emb3, emb4),
                        _TBASE, _TBASE[1:] + [_TPACK]):
        r = e.reshape(-1)
        pieces.append(r)
        pad = nb - b - r.shape[0]
        if pad:
            pieces.append(jnp.zeros((pad,), jnp.float32))
    tbl = jnp.concatenate(pieces)

    mesh = plsc.VectorSubcoreMesh(core_axis_name="c", subcore_axis_name="s")
    kern = pl.kernel(
        _body,
        out_type=jax.ShapeDtypeStruct((_OUT_W, _B), jnp.float32),
        mesh=mesh,
        scratch_types=[
            pltpu.VMEM((_TPACK,), jnp.float32),
            pltpu.VMEM((5 * _BPW,), jnp.int32),
            pltpu.VMEM((_OUT_W, _BPW), jnp.float32),
            pltpu.SemaphoreType.DMA,
            pltpu.SemaphoreType.DMA,
            pltpu.SemaphoreType.DMA,
        ],
        compiler_params=pltpu.CompilerParams(
            needs_layout_passes=False, use_tc_tiling_on_sc=False),
    )
    out_t = kern(xt, yw, tbl)
    return out_t.T
